# Initial kernel scaffold; baseline (speedup 1.0000x reference)
import jax
import jax.numpy as jnp
from jax.experimental import pallas as pl


def kernel(x, edge_index, edge_weight):
    raise NotImplementedError("write your pallas kernel here")


# speedup vs baseline: 157.6736x; 157.6736x over previous
import jax
import jax.numpy as jnp
from jax import lax
from jax.experimental import pallas as pl
from jax.experimental.pallas import tpu as pltpu
from jax.experimental.pallas import tpu_sc as plsc

N = 100000
E = 1600000
NUM_ITERS = 50

NTILES = 16
LANES = 16

SLICE = 6400
N_PAD = SLICE * NTILES
W = 2048
CHUNKS = W // 128
NWIN = 51
EPT = NWIN * W
E_PAD = EPT * NTILES
ROWS_PT = EPT // 128
ZCHUNK = 1600
VECS = 128 // LANES
NBUF = 3


def _body(x_hbm, dst_hbm, wt_hbm, src_hbm, out_hbm,
          dinv_buf, out_buf, agg_buf, zero_buf,
          dst_win, wt_win, src_win, gath, msg,
          w_spmem, agg_spmem,
          load_sem, gath_sem, scat_sem):
    c = lax.axis_index("c")
    s = lax.axis_index("s")
    slice0 = s * SLICE
    base_row = s * ROWS_PT

    zeros16 = jnp.zeros((LANES,), jnp.float32)

    @pl.loop(0, ZCHUNK // LANES)
    def _(i):
        zero_buf[pl.ds(i * LANES, LANES)] = zeros16

    @pl.loop(0, SLICE // LANES)
    def _(i):
        out_buf[pl.ds(i * LANES, LANES)] = zeros16

    pltpu.sync_copy(x_hbm.at[pl.ds(slice0, SLICE)],
                    w_spmem.at[pl.ds(slice0, SLICE)])
    for q in range(SLICE // ZCHUNK):
        pltpu.sync_copy(zero_buf,
                        agg_spmem.at[pl.ds(slice0 + q * ZCHUNK, ZCHUNK)])
    plsc.subcore_barrier()

    def fire_loads(i, r, need_gather):
        row0 = base_row + i * CHUNKS
        pltpu.async_copy(wt_hbm.at[pl.ds(row0, CHUNKS)], wt_win.at[r], load_sem)
        pltpu.async_copy(src_hbm.at[pl.ds(row0, CHUNKS)], src_win.at[r],
                         load_sem)
        if need_gather:
            pltpu.async_copy(dst_hbm.at[pl.ds(row0, CHUNKS)], dst_win.at[r],
                             load_sem)

    def wait_loads(i, r, need_gather):
        row0 = base_row + i * CHUNKS
        pltpu.make_async_copy(wt_hbm.at[pl.ds(row0, CHUNKS)], wt_win.at[r],
                              load_sem).wait()
        pltpu.make_async_copy(src_hbm.at[pl.ds(row0, CHUNKS)], src_win.at[r],
                              load_sem).wait()
        if need_gather:
            pltpu.make_async_copy(dst_hbm.at[pl.ds(row0, CHUNKS)],
                                  dst_win.at[r], load_sem).wait()

    def fire_gathers(r):
        for j in range(CHUNKS):
            pltpu.async_copy(w_spmem.at[dst_win.at[r, j]], gath.at[r, j],
                             gath_sem)

    def wait_gathers(r):
        for j in range(CHUNKS):
            pltpu.make_async_copy(w_spmem.at[dst_win.at[r, j]], gath.at[r, j],
                                  gath_sem).wait()

    def fire_scatters(r, need_gather):
        vals = msg if need_gather else wt_win
        for j in range(CHUNKS):
            pltpu.async_copy(vals.at[r, j], agg_spmem.at[src_win.at[r, j]],
                             scat_sem, add=True)

    def drain_scatters(r, need_gather):
        vals = msg if need_gather else wt_win
        for j in range(CHUNKS):
            pltpu.make_async_copy(vals.at[r, j],
                                  agg_spmem.at[src_win.at[r, j]],
                                  scat_sem).wait()

    def compute(r):
        for j in range(CHUNKS):
            for i in range(VECS):
                wv = wt_win[r, j, pl.ds(i * LANES, LANES)]
                gv = gath[r, j, pl.ds(i * LANES, LANES)]
                msg[r, j, pl.ds(i * LANES, LANES)] = wv * gv

    def edge_pass(need_gather):
        fire_loads(0, 0, need_gather)
        fire_loads(1, 1, need_gather)
        wait_loads(0, 0, need_gather)
        if need_gather:
            fire_gathers(0)

        @pl.loop(0, NWIN // NBUF)
        def _(t):
            for r in range(NBUF):
                i = t * NBUF + r
                rn = (r + 1) % NBUF
                rp = (r + 2) % NBUF

                @pl.when(i + 1 < NWIN)
                def _():
                    wait_loads(i + 1, rn, need_gather)
                    if need_gather:
                        fire_gathers(rn)

                @pl.when(i > 0)
                def _():
                    drain_scatters(rp, need_gather)

                @pl.when(i + 2 < NWIN)
                def _():
                    fire_loads(i + 2, rp, need_gather)

                if need_gather:
                    wait_gathers(r)
                    compute(r)
                fire_scatters(r, need_gather)

        drain_scatters((NWIN - 1) % NBUF, need_gather)

    edge_pass(False)
    plsc.subcore_barrier()
    pltpu.sync_copy(agg_spmem.at[pl.ds(slice0, SLICE)], agg_buf)

    @pl.loop(0, SLICE // LANES)
    def _(i):
        d = agg_buf[pl.ds(i * LANES, LANES)]
        dinv_buf[pl.ds(i * LANES, LANES)] = jnp.where(d > 0.0, 1.0 / d, 0.0)

    for q in range(SLICE // ZCHUNK):
        pltpu.sync_copy(zero_buf,
                        agg_spmem.at[pl.ds(slice0 + q * ZCHUNK, ZCHUNK)])
    plsc.subcore_barrier()

    @pl.loop(0, NUM_ITERS)
    def _(k):
        edge_pass(True)
        plsc.subcore_barrier()
        pltpu.sync_copy(agg_spmem.at[pl.ds(slice0, SLICE)], agg_buf)

        @pl.loop(0, SLICE // LANES)
        def _(i):
            a = agg_buf[pl.ds(i * LANES, LANES)]
            di = dinv_buf[pl.ds(i * LANES, LANES)]
            wv = a * di
            out_buf[pl.ds(i * LANES, LANES)] = (
                out_buf[pl.ds(i * LANES, LANES)] + wv)
            agg_buf[pl.ds(i * LANES, LANES)] = wv

        pltpu.sync_copy(agg_buf, w_spmem.at[pl.ds(slice0, SLICE)])
        for q in range(SLICE // ZCHUNK):
            pltpu.sync_copy(zero_buf,
                            agg_spmem.at[pl.ds(slice0 + q * ZCHUNK, ZCHUNK)])
        plsc.subcore_barrier()

    @pl.when(c == 0)
    def _():
        pltpu.sync_copy(out_buf, out_hbm.at[pl.ds(slice0, SLICE)])


_sc_kernel = pl.kernel(
    _body,
    out_type=jax.ShapeDtypeStruct((N_PAD,), jnp.float32),
    mesh=plsc.VectorSubcoreMesh(core_axis_name="c", subcore_axis_name="s"),
    compiler_params=pltpu.CompilerParams(needs_layout_passes=False),
    scratch_types=[
        pltpu.VMEM((SLICE,), jnp.float32),
        pltpu.VMEM((SLICE,), jnp.float32),
        pltpu.VMEM((SLICE,), jnp.float32),
        pltpu.VMEM((ZCHUNK,), jnp.float32),
        pltpu.VMEM((NBUF, CHUNKS, 128), jnp.int32),
        pltpu.VMEM((NBUF, CHUNKS, 128), jnp.float32),
        pltpu.VMEM((NBUF, CHUNKS, 128), jnp.int32),
        pltpu.VMEM((NBUF, CHUNKS, 128), jnp.float32),
        pltpu.VMEM((NBUF, CHUNKS, 128), jnp.float32),
        pltpu.VMEM_SHARED((N_PAD,), jnp.float32),
        pltpu.VMEM_SHARED((N_PAD,), jnp.float32),
        pltpu.SemaphoreType.DMA,
        pltpu.SemaphoreType.DMA,
        pltpu.SemaphoreType.DMA,
    ],
)


def kernel(x, edge_index, edge_weight):
    src = edge_index[0].astype(jnp.int32)
    dst = edge_index[1].astype(jnp.int32)
    wt = edge_weight.astype(jnp.float32)

    pad = E_PAD - E
    spread = jnp.arange(pad, dtype=jnp.int32) % N
    src_p = jnp.concatenate([src, spread])
    dst_p = jnp.concatenate([dst, spread])
    wt_p = jnp.concatenate([wt, jnp.zeros((pad,), jnp.float32)])

    x_p = jnp.pad(x[:, 0].astype(jnp.float32), (0, N_PAD - N))

    out = _sc_kernel(x_p,
                     dst_p.reshape(E_PAD // 128, 128),
                     wt_p.reshape(E_PAD // 128, 128),
                     src_p.reshape(E_PAD // 128, 128))
    return out[:N, None]


# speedup vs baseline: 224.5372x; 1.4241x over previous
import jax
import jax.numpy as jnp
from jax import lax
from jax.experimental import pallas as pl
from jax.experimental.pallas import tpu as pltpu
from jax.experimental.pallas import tpu_sc as plsc

N = 100000
E = 1600000
NUM_ITERS = 50

NTILES = 16
LANES = 16

SLICE = 6400
N_PAD = SLICE * NTILES
W = 1024
CHUNKS = W // 128
NWIN = 102
EPT = NWIN * W
E_PAD = EPT * NTILES
ROWS_PT = EPT // 128
CCH = 1600
NCH = SLICE // CCH
ZLEN = 800
VECS = 128 // LANES
NBUF = 3


def _body(x_hbm, dst_hbm, wt_hbm, src_hbm,
          out_hbm, wpub0, wpub1,
          w_full, dinv_buf, cagg, cout, zero_buf,
          dst_win, wt_win, src_win, msg,
          agg_spmem,
          load_sem, scat_sem):
    c = lax.axis_index("c")
    s = lax.axis_index("s")
    slice0 = s * SLICE
    base_row = s * ROWS_PT

    zeros16 = jnp.zeros((LANES,), jnp.float32)

    def on_my_core(fn):
        @pl.when(c == 0)
        def _():
            fn(wpub0)

        @pl.when(c == 1)
        def _():
            fn(wpub1)

    @pl.loop(0, ZLEN // LANES)
    def _(i):
        zero_buf[pl.ds(i * LANES, LANES)] = zeros16

    for q in range(NCH):
        pltpu.sync_copy(x_hbm.at[pl.ds(slice0 + q * CCH, CCH)], cagg)

        def _stage(wpub, _q=q):
            pltpu.sync_copy(cagg, wpub.at[pl.ds(slice0 + _q * CCH, CCH)])
        on_my_core(_stage)
        for z in range(CCH // ZLEN):
            pltpu.sync_copy(
                zero_buf,
                agg_spmem.at[pl.ds(slice0 + q * CCH + z * ZLEN, ZLEN)])

    @pl.when(c == 0)
    def _():
        for z in range(SLICE // ZLEN):
            pltpu.sync_copy(zero_buf,
                            out_hbm.at[pl.ds(slice0 + z * ZLEN, ZLEN)])

    plsc.subcore_barrier()

    def fire_loads(i, r, need_gather):
        row0 = base_row + i * CHUNKS
        pltpu.async_copy(wt_hbm.at[pl.ds(row0, CHUNKS)], wt_win.at[r], load_sem)
        pltpu.async_copy(src_hbm.at[pl.ds(row0, CHUNKS)], src_win.at[r],
                         load_sem)
        if need_gather:
            pltpu.async_copy(dst_hbm.at[pl.ds(row0, CHUNKS)], dst_win.at[r],
                             load_sem)

    def wait_loads(i, r, need_gather):
        row0 = base_row + i * CHUNKS
        pltpu.make_async_copy(wt_hbm.at[pl.ds(row0, CHUNKS)], wt_win.at[r],
                              load_sem).wait()
        pltpu.make_async_copy(src_hbm.at[pl.ds(row0, CHUNKS)], src_win.at[r],
                              load_sem).wait()
        if need_gather:
            pltpu.make_async_copy(dst_hbm.at[pl.ds(row0, CHUNKS)],
                                  dst_win.at[r], load_sem).wait()

    def fire_scatters(r, need_gather):
        vals = msg if need_gather else wt_win
        for j in range(CHUNKS):
            pltpu.async_copy(vals.at[r, j], agg_spmem.at[src_win.at[r, j]],
                             scat_sem, add=True)

    def drain_scatters(r, need_gather):
        vals = msg if need_gather else wt_win
        for j in range(CHUNKS):
            pltpu.make_async_copy(vals.at[r, j],
                                  agg_spmem.at[src_win.at[r, j]],
                                  scat_sem).wait()

    def compute(r):
        for j in range(CHUNKS):
            for i in range(VECS):
                dv = dst_win[r, j, pl.ds(i * LANES, LANES)]
                wv = wt_win[r, j, pl.ds(i * LANES, LANES)]
                gv = plsc.load_gather(w_full, [dv])
                msg[r, j, pl.ds(i * LANES, LANES)] = wv * gv

    def edge_pass(need_gather):
        fire_loads(0, 0, need_gather)
        fire_loads(1, 1, need_gather)

        @pl.loop(0, NWIN // NBUF)
        def _(t):
            for r in range(NBUF):
                i = t * NBUF + r
                rp = (r + 2) % NBUF

                wait_loads(i, r, need_gather)
                if need_gather:
                    compute(r)

                @pl.when(i > 0)
                def _():
                    drain_scatters(rp, need_gather)

                @pl.when(i + 2 < NWIN)
                def _():
                    fire_loads(i + 2, rp, need_gather)

                fire_scatters(r, need_gather)

        drain_scatters((NWIN - 1) % NBUF, need_gather)

    edge_pass(False)
    plsc.subcore_barrier()
    for q in range(NCH):
        pltpu.sync_copy(agg_spmem.at[pl.ds(slice0 + q * CCH, CCH)], cagg)

        @pl.loop(0, CCH // LANES)
        def _(i):
            d = cagg[pl.ds(i * LANES, LANES)]
            dinv_buf[pl.ds(q * CCH + i * LANES, LANES)] = (
                jnp.where(d > 0.0, 1.0 / d, 0.0))

        for z in range(CCH // ZLEN):
            pltpu.sync_copy(
                zero_buf,
                agg_spmem.at[pl.ds(slice0 + q * CCH + z * ZLEN, ZLEN)])
    plsc.subcore_barrier()

    @pl.loop(0, NUM_ITERS)
    def _(k):
        def _reload(wpub):
            pltpu.sync_copy(wpub.at[pl.ds(0, N)], w_full)
        on_my_core(_reload)
        edge_pass(True)
        plsc.subcore_barrier()
        for q in range(NCH):
            off = slice0 + q * CCH
            pltpu.sync_copy(agg_spmem.at[pl.ds(off, CCH)], cagg)

            @pl.loop(0, CCH // LANES)
            def _(i):
                a = cagg[pl.ds(i * LANES, LANES)]
                di = dinv_buf[pl.ds(q * CCH + i * LANES, LANES)]
                cagg[pl.ds(i * LANES, LANES)] = a * di

            def _pub(wpub, _off=off):
                pltpu.sync_copy(cagg, wpub.at[pl.ds(_off, CCH)])
            on_my_core(_pub)

            @pl.when(c == 0)
            def _():
                pltpu.sync_copy(out_hbm.at[pl.ds(off, CCH)], cout)

                @pl.loop(0, CCH // LANES)
                def _(i):
                    cout[pl.ds(i * LANES, LANES)] = (
                        cout[pl.ds(i * LANES, LANES)]
                        + cagg[pl.ds(i * LANES, LANES)])

                pltpu.sync_copy(cout, out_hbm.at[pl.ds(off, CCH)])

            for z in range(CCH // ZLEN):
                pltpu.sync_copy(zero_buf,
                                agg_spmem.at[pl.ds(off + z * ZLEN, ZLEN)])
        plsc.subcore_barrier()


_sc_kernel = pl.kernel(
    _body,
    out_type=(
        jax.ShapeDtypeStruct((N_PAD,), jnp.float32),
        jax.ShapeDtypeStruct((N_PAD,), jnp.float32),
        jax.ShapeDtypeStruct((N_PAD,), jnp.float32),
    ),
    mesh=plsc.VectorSubcoreMesh(core_axis_name="c", subcore_axis_name="s"),
    compiler_params=pltpu.CompilerParams(needs_layout_passes=False),
    scratch_types=[
        pltpu.VMEM((N,), jnp.float32),
        pltpu.VMEM((SLICE,), jnp.float32),
        pltpu.VMEM((CCH,), jnp.float32),
        pltpu.VMEM((CCH,), jnp.float32),
        pltpu.VMEM((ZLEN,), jnp.float32),
        pltpu.VMEM((NBUF, CHUNKS, 128), jnp.int32),
        pltpu.VMEM((NBUF, CHUNKS, 128), jnp.float32),
        pltpu.VMEM((NBUF, CHUNKS, 128), jnp.int32),
        pltpu.VMEM((NBUF, CHUNKS, 128), jnp.float32),
        pltpu.VMEM_SHARED((N_PAD,), jnp.float32),
        pltpu.SemaphoreType.DMA,
        pltpu.SemaphoreType.DMA,
    ],
)


def kernel(x, edge_index, edge_weight):
    src = edge_index[0].astype(jnp.int32)
    dst = edge_index[1].astype(jnp.int32)
    wt = edge_weight.astype(jnp.float32)

    pad = E_PAD - E
    spread = jnp.arange(pad, dtype=jnp.int32) % N
    src_p = jnp.concatenate([src, spread])
    dst_p = jnp.concatenate([dst, spread])
    wt_p = jnp.concatenate([wt, jnp.zeros((pad,), jnp.float32)])

    x_p = jnp.pad(x[:, 0].astype(jnp.float32), (0, N_PAD - N))

    out, _, _ = _sc_kernel(x_p,
                           dst_p.reshape(E_PAD // 128, 128),
                           wt_p.reshape(E_PAD // 128, 128),
                           src_p.reshape(E_PAD // 128, 128))
    return out[:N, None]
